# Initial kernel scaffold; baseline (speedup 1.0000x reference)
import jax
import jax.numpy as jnp
from jax.experimental import pallas as pl


def kernel(x, edge_index, W1, b1, W2, b2):
    raise NotImplementedError("write your pallas kernel here")


# speedup vs baseline: 21.1837x; 21.1837x over previous
import functools

import jax
import jax.numpy as jnp
from jax import lax
from jax.experimental import pallas as pl
from jax.experimental.pallas import tpu as pltpu
from jax.experimental.pallas import tpu_sc as plsc

N = 10000
E = 320000
NC = 2
NS = 16
NW = NC * NS
C = 128
K = 80
EP = NW * K * C
NP = 10240
ROWS_PER_TILE = NP // NS
DEG_W = 8

_BLK = 1000
_GRID = N // _BLK


def _sc_degree_body(dst_hbm, ones_hbm, zeros_hbm, out_hbm, dst_v, ones_v, acc):
    c = lax.axis_index("c")
    s = lax.axis_index("s")
    wid = s * NC + c
    row0 = s * ROWS_PER_TILE
    pltpu.sync_copy(zeros_hbm.at[pl.ds(row0, ROWS_PER_TILE)],
                    acc.at[pl.ds(row0, ROWS_PER_TILE)])
    pltpu.sync_copy(dst_hbm.at[wid], dst_v)
    pltpu.sync_copy(ones_hbm, ones_v)
    plsc.subcore_barrier()

    def step(j, carry):
        pltpu.sync_copy(ones_v, acc.at[dst_v.at[j]], add=True)
        return carry

    lax.fori_loop(0, K, step, 0)
    plsc.subcore_barrier()
    pltpu.sync_copy(acc.at[pl.ds(row0, ROWS_PER_TILE)],
                    out_hbm.at[c, pl.ds(row0, ROWS_PER_TILE)])


def _sc_spmm_body(ys_hbm, src_hbm, dst_hbm, zeros_hbm, out_hbm,
                  src_v, dst_v, rows_v, acc, sem):
    c = lax.axis_index("c")
    s = lax.axis_index("s")
    wid = s * NC + c
    row0 = s * ROWS_PER_TILE
    pltpu.sync_copy(zeros_hbm.at[pl.ds(row0, ROWS_PER_TILE)],
                    acc.at[pl.ds(row0, ROWS_PER_TILE)])
    pltpu.sync_copy(src_hbm.at[wid], src_v)
    pltpu.sync_copy(dst_hbm.at[wid], dst_v)
    plsc.subcore_barrier()

    def step(j, carry):
        pltpu.async_copy(ys_hbm.at[src_v.at[j]], rows_v, sem).wait()
        pltpu.sync_copy(rows_v, acc.at[dst_v.at[j]], add=True)
        return carry

    lax.fori_loop(0, K, step, 0)
    plsc.subcore_barrier()
    pltpu.sync_copy(acc.at[pl.ds(row0, ROWS_PER_TILE)],
                    out_hbm.at[c, pl.ds(row0, ROWS_PER_TILE)])


def _sc_degree(dst3):
    mesh = plsc.VectorSubcoreMesh(core_axis_name="c", subcore_axis_name="s")
    ones = jnp.ones((C,), jnp.float32)
    zeros = jnp.zeros((NP,), jnp.float32)
    k = pl.kernel(
        _sc_degree_body,
        out_type=jax.ShapeDtypeStruct((NC, NP), jnp.float32),
        mesh=mesh,
        scratch_types=[
            pltpu.VMEM((K, C), jnp.int32),
            pltpu.VMEM((C,), jnp.float32),
            pltpu.VMEM_SHARED((NP,), jnp.float32),
        ],
    )
    return k(dst3, ones, zeros)


def _sc_spmm(ys, src3, dst3, d):
    mesh = plsc.VectorSubcoreMesh(core_axis_name="c", subcore_axis_name="s")
    zeros = jnp.zeros((NP, d), jnp.float32)
    k = pl.kernel(
        _sc_spmm_body,
        out_type=jax.ShapeDtypeStruct((NC, NP, d), jnp.float32),
        mesh=mesh,
        scratch_types=[
            pltpu.VMEM((K, C), jnp.int32),
            pltpu.VMEM((K, C), jnp.int32),
            pltpu.VMEM((C, d), jnp.float32),
            pltpu.VMEM_SHARED((NP, d), jnp.float32),
            pltpu.SemaphoreType.DMA,
        ],
    )
    return k(ys, src3, dst3, zeros)


def _dinv(d0_ref, d1_ref):
    deg = d0_ref[...] + d1_ref[...] + 1.0
    return lax.rsqrt(deg)


def _tc_mm_scale_body(x_ref, w_ref, d0_ref, d1_ref, o_ref):
    y = jnp.dot(x_ref[...], w_ref[...], preferred_element_type=jnp.float32)
    o_ref[...] = y * _dinv(d0_ref, d1_ref)


def _tc_mid_body(p0_ref, p1_ref, ys_ref, d0_ref, d1_ref, b_ref, w_ref, o_ref):
    dinv = _dinv(d0_ref, d1_ref)
    h = jnp.maximum(dinv * (p0_ref[...] + p1_ref[...] + ys_ref[...])
                    + b_ref[...], 0.0)
    o_ref[...] = jnp.dot(h, w_ref[...], preferred_element_type=jnp.float32) * dinv


def _tc_final_body(q0_ref, q1_ref, ys_ref, d0_ref, d1_ref, b_ref, o_ref):
    dinv = _dinv(d0_ref, d1_ref)
    full = dinv * (q0_ref[...] + q1_ref[...] + ys_ref[...])
    o_ref[...] = full[:, :o_ref.shape[1]] + b_ref[...]


def _row_blocked(d):
    return pl.BlockSpec((_BLK, d), lambda i: (i, 0))


def _full(shape):
    return pl.BlockSpec(shape, lambda i: tuple(0 for _ in shape))


def kernel(x, edge_index, W1, b1, W2, b2):
    npad = EP - E
    pad_src = (jnp.arange(npad, dtype=jnp.int32) * 37) % N
    pad_dst = N + (jnp.arange(npad, dtype=jnp.int32) % (NP - N))
    src3 = jnp.concatenate([edge_index[0], pad_src]).reshape(NW, K, C)
    dst3 = jnp.concatenate([edge_index[1], pad_dst]).reshape(NW, K, C)
    d_hid = W1.shape[1]
    d_out = W2.shape[1]

    degp = _sc_degree(dst3)
    d0 = degp[0].reshape(NP, 1)
    d1 = degp[1].reshape(NP, 1)

    ys1 = pl.pallas_call(
        _tc_mm_scale_body,
        grid=(_GRID,),
        in_specs=[_row_blocked(x.shape[1]), _full(W1.shape),
                  _row_blocked(1), _row_blocked(1)],
        out_specs=_row_blocked(d_hid),
        out_shape=jax.ShapeDtypeStruct((N, d_hid), jnp.float32),
    )(x, W1, d0, d1)

    p = _sc_spmm(ys1, src3, dst3, d_hid)

    W2p = jnp.zeros((d_hid, d_hid), jnp.float32).at[:, :d_out].set(W2)

    ys2 = pl.pallas_call(
        _tc_mid_body,
        grid=(_GRID,),
        in_specs=[_row_blocked(d_hid), _row_blocked(d_hid), _row_blocked(d_hid),
                  _row_blocked(1), _row_blocked(1),
                  _full((1, d_hid)), _full(W2p.shape)],
        out_specs=_row_blocked(d_hid),
        out_shape=jax.ShapeDtypeStruct((N, d_hid), jnp.float32),
    )(p[0], p[1], ys1, d0, d1, b1.reshape(1, d_hid), W2p)

    q = _sc_spmm(ys2, src3, dst3, d_hid)

    out = pl.pallas_call(
        _tc_final_body,
        grid=(_GRID,),
        in_specs=[_row_blocked(d_hid), _row_blocked(d_hid), _row_blocked(d_hid),
                  _row_blocked(1), _row_blocked(1), _full((1, d_out))],
        out_specs=_row_blocked(d_out),
        out_shape=jax.ShapeDtypeStruct((N, d_out), jnp.float32),
    )(q[0], q[1], ys2, d0, d1, b2.reshape(1, d_out))

    return out


# speedup vs baseline: 29.6127x; 1.3979x over previous
import functools

import jax
import jax.numpy as jnp
from jax import lax
from jax.experimental import pallas as pl
from jax.experimental.pallas import tpu as pltpu
from jax.experimental.pallas import tpu_sc as plsc

N = 10000
E = 320000
NC = 2
NS = 16
NW = NC * NS
C = 128
K = 80
EP = NW * K * C
NP = 10240
ROWS_PER_TILE = NP // NS
DEG_W = 8

_BLK = 1000
_GRID = N // _BLK


def _sc_degree_body(dst_hbm, ones_hbm, zeros_hbm, out_hbm, dst_v, ones_v, acc):
    c = lax.axis_index("c")
    s = lax.axis_index("s")
    wid = s * NC + c
    row0 = s * ROWS_PER_TILE
    pltpu.sync_copy(zeros_hbm.at[pl.ds(row0, ROWS_PER_TILE)],
                    acc.at[pl.ds(row0, ROWS_PER_TILE)])
    pltpu.sync_copy(dst_hbm.at[wid], dst_v)
    pltpu.sync_copy(ones_hbm, ones_v)
    plsc.subcore_barrier()

    def step(j, carry):
        pltpu.sync_copy(ones_v, acc.at[dst_v.at[j]], add=True)
        return carry

    lax.fori_loop(0, K, step, 0)
    plsc.subcore_barrier()
    pltpu.sync_copy(acc.at[pl.ds(row0, ROWS_PER_TILE)],
                    out_hbm.at[c, pl.ds(row0, ROWS_PER_TILE)])


def _sc_spmm_body(ys_hbm, src_hbm, dst_hbm, zeros_hbm, out_hbm,
                  src_v, dst_v, rows_a, rows_b, acc, sem_a, sem_b):
    c = lax.axis_index("c")
    s = lax.axis_index("s")
    wid = s * NC + c
    row0 = s * ROWS_PER_TILE
    pltpu.sync_copy(zeros_hbm.at[pl.ds(row0, ROWS_PER_TILE)],
                    acc.at[pl.ds(row0, ROWS_PER_TILE)])
    plsc.subcore_barrier()

    def gather(j, buf, sem):
        return pltpu.make_async_copy(ys_hbm.at[src_v.at[j]], buf, sem)

    KH = K // 2
    for h in range(2):
        pltpu.sync_copy(src_hbm.at[wid, pl.ds(h * KH, KH)], src_v)
        pltpu.sync_copy(dst_hbm.at[wid, pl.ds(h * KH, KH)], dst_v)
        gather(0, rows_a, sem_a).start()

        def step(jj, carry):
            ja = 2 * jj
            gather(ja + 1, rows_b, sem_b).start()
            gather(ja, rows_a, sem_a).wait()
            pltpu.sync_copy(rows_a, acc.at[dst_v.at[ja]], add=True)

            @pl.when(jj < KH // 2 - 1)
            def _():
                gather(ja + 2, rows_a, sem_a).start()

            gather(ja + 1, rows_b, sem_b).wait()
            pltpu.sync_copy(rows_b, acc.at[dst_v.at[ja + 1]], add=True)
            return carry

        lax.fori_loop(0, KH // 2, step, 0)
    plsc.subcore_barrier()
    pltpu.sync_copy(acc.at[pl.ds(row0, ROWS_PER_TILE)],
                    out_hbm.at[c, pl.ds(row0, ROWS_PER_TILE)])


def _sc_degree(dst3):
    mesh = plsc.VectorSubcoreMesh(core_axis_name="c", subcore_axis_name="s")
    ones = jnp.ones((C,), jnp.float32)
    zeros = jnp.zeros((NP,), jnp.float32)
    k = pl.kernel(
        _sc_degree_body,
        out_type=jax.ShapeDtypeStruct((NC, NP), jnp.float32),
        mesh=mesh,
        scratch_types=[
            pltpu.VMEM((K, C), jnp.int32),
            pltpu.VMEM((C,), jnp.float32),
            pltpu.VMEM_SHARED((NP,), jnp.float32),
        ],
    )
    return k(dst3, ones, zeros)


def _sc_spmm(ys, src3, dst3, d):
    mesh = plsc.VectorSubcoreMesh(core_axis_name="c", subcore_axis_name="s")
    zeros = jnp.zeros((NP, d), jnp.float32)
    k = pl.kernel(
        _sc_spmm_body,
        out_type=jax.ShapeDtypeStruct((NC, NP, d), jnp.float32),
        mesh=mesh,
        scratch_types=[
            pltpu.VMEM((K // 2, C), jnp.int32),
            pltpu.VMEM((K // 2, C), jnp.int32),
            pltpu.VMEM((C, d), jnp.float32),
            pltpu.VMEM((C, d), jnp.float32),
            pltpu.VMEM_SHARED((NP, d), jnp.float32),
            pltpu.SemaphoreType.DMA,
            pltpu.SemaphoreType.DMA,
        ],
    )
    return k(ys, src3, dst3, zeros)


def _dinv(d0_ref, d1_ref):
    deg = d0_ref[...] + d1_ref[...] + 1.0
    return lax.rsqrt(deg)


def _tc_mm_scale_body(x_ref, w_ref, d0_ref, d1_ref, o_ref):
    y = jnp.dot(x_ref[...], w_ref[...], preferred_element_type=jnp.float32)
    o_ref[...] = y * _dinv(d0_ref, d1_ref)


def _tc_mid_body(p0_ref, p1_ref, ys_ref, d0_ref, d1_ref, b_ref, w_ref, o_ref):
    dinv = _dinv(d0_ref, d1_ref)
    h = jnp.maximum(dinv * (p0_ref[...] + p1_ref[...] + ys_ref[...])
                    + b_ref[...], 0.0)
    o_ref[...] = jnp.dot(h, w_ref[...], preferred_element_type=jnp.float32) * dinv


def _tc_final_body(q0_ref, q1_ref, ys_ref, d0_ref, d1_ref, b_ref, o_ref):
    dinv = _dinv(d0_ref, d1_ref)
    full = dinv * (q0_ref[...] + q1_ref[...] + ys_ref[...])
    o_ref[...] = full[:, :o_ref.shape[1]] + b_ref[...]


def _row_blocked(d):
    return pl.BlockSpec((_BLK, d), lambda i: (i, 0))


def _full(shape):
    return pl.BlockSpec(shape, lambda i: tuple(0 for _ in shape))


def kernel(x, edge_index, W1, b1, W2, b2):
    npad = EP - E
    pad_src = (jnp.arange(npad, dtype=jnp.int32) * 37) % N
    pad_dst = N + (jnp.arange(npad, dtype=jnp.int32) % (NP - N))
    src3 = jnp.concatenate([edge_index[0], pad_src]).reshape(NW, K, C)
    dst3 = jnp.concatenate([edge_index[1], pad_dst]).reshape(NW, K, C)
    d_hid = W1.shape[1]
    d_out = W2.shape[1]

    degp = _sc_degree(dst3)
    d0 = degp[0].reshape(NP, 1)
    d1 = degp[1].reshape(NP, 1)

    ys1 = pl.pallas_call(
        _tc_mm_scale_body,
        grid=(_GRID,),
        in_specs=[_row_blocked(x.shape[1]), _full(W1.shape),
                  _row_blocked(1), _row_blocked(1)],
        out_specs=_row_blocked(d_hid),
        out_shape=jax.ShapeDtypeStruct((N, d_hid), jnp.float32),
    )(x, W1, d0, d1)

    p = _sc_spmm(ys1, src3, dst3, d_hid)

    W2p = jnp.zeros((d_hid, d_hid), jnp.float32).at[:, :d_out].set(W2)

    ys2 = pl.pallas_call(
        _tc_mid_body,
        grid=(_GRID,),
        in_specs=[_row_blocked(d_hid), _row_blocked(d_hid), _row_blocked(d_hid),
                  _row_blocked(1), _row_blocked(1),
                  _full((1, d_hid)), _full(W2p.shape)],
        out_specs=_row_blocked(d_hid),
        out_shape=jax.ShapeDtypeStruct((N, d_hid), jnp.float32),
    )(p[0], p[1], ys1, d0, d1, b1.reshape(1, d_hid), W2p)

    q = _sc_spmm(ys2, src3, dst3, d_hid)

    out = pl.pallas_call(
        _tc_final_body,
        grid=(_GRID,),
        in_specs=[_row_blocked(d_hid), _row_blocked(d_hid), _row_blocked(d_hid),
                  _row_blocked(1), _row_blocked(1), _full((1, d_out))],
        out_specs=_row_blocked(d_out),
        out_shape=jax.ShapeDtypeStruct((N, d_out), jnp.float32),
    )(q[0], q[1], ys2, d0, d1, b2.reshape(1, d_out))

    return out


# speedup vs baseline: 30.7565x; 1.0386x over previous
import jax
import jax.numpy as jnp
from jax import lax
from jax.experimental import pallas as pl
from jax.experimental.pallas import tpu as pltpu
from jax.experimental.pallas import tpu_sc as plsc

N = 10000
E = 320000
NC = 2
NS = 16
NW = NC * NS
C = 128
K = 80
EP = NW * K * C
NP = 10240
ROWS_PER_TILE = NP // NS
D = 128

_BLK = 1000
_GRID = N // _BLK


def _sc_degree_body(dst_hbm, ones_hbm, zeros_hbm, out_hbm, dst_v, ones_v, acc):
    c = lax.axis_index("c")
    s = lax.axis_index("s")
    wid = s * NC + c
    row0 = s * ROWS_PER_TILE
    pltpu.sync_copy(zeros_hbm.at[pl.ds(row0, ROWS_PER_TILE)],
                    acc.at[pl.ds(row0, ROWS_PER_TILE)])
    pltpu.sync_copy(dst_hbm.at[wid], dst_v)
    pltpu.sync_copy(ones_hbm, ones_v)
    plsc.subcore_barrier()

    def step(j, carry):
        pltpu.sync_copy(ones_v, acc.at[dst_v.at[j]], add=True)
        return carry

    lax.fori_loop(0, K, step, 0)
    plsc.subcore_barrier()
    pltpu.sync_copy(acc.at[pl.ds(row0, ROWS_PER_TILE)],
                    out_hbm.at[c, pl.ds(row0, ROWS_PER_TILE)])


def _sc_spmm_body(ys_hbm, src_hbm, dst_hbm, zeros_hbm, out_hbm,
                  src_v, dst_v, rows_a, rows_b, acc, sem_a, sem_b):
    c = lax.axis_index("c")
    s = lax.axis_index("s")
    wid = s * NC + c
    row0 = s * ROWS_PER_TILE
    pltpu.sync_copy(zeros_hbm.at[pl.ds(row0, ROWS_PER_TILE)],
                    acc.at[pl.ds(row0, ROWS_PER_TILE)])
    plsc.subcore_barrier()

    def gather(j, buf, sem):
        return pltpu.make_async_copy(ys_hbm.at[src_v.at[j]], buf, sem)

    KH = K // 2
    for h in range(2):
        pltpu.sync_copy(src_hbm.at[wid, pl.ds(h * KH, KH)], src_v)
        pltpu.sync_copy(dst_hbm.at[wid, pl.ds(h * KH, KH)], dst_v)
        gather(0, rows_a, sem_a).start()

        def step(jj, carry):
            ja = 2 * jj
            gather(ja + 1, rows_b, sem_b).start()
            gather(ja, rows_a, sem_a).wait()
            pltpu.sync_copy(rows_a, acc.at[dst_v.at[ja]], add=True)

            @pl.when(jj < KH // 2 - 1)
            def _():
                gather(ja + 2, rows_a, sem_a).start()

            gather(ja + 1, rows_b, sem_b).wait()
            pltpu.sync_copy(rows_b, acc.at[dst_v.at[ja + 1]], add=True)
            return carry

        lax.fori_loop(0, KH // 2, step, 0)
    plsc.subcore_barrier()
    pltpu.sync_copy(acc.at[pl.ds(row0, ROWS_PER_TILE)],
                    out_hbm.at[pl.ds(row0, ROWS_PER_TILE), pl.ds(c * D, D)])


def _sc_degree(dst3):
    mesh = plsc.VectorSubcoreMesh(core_axis_name="c", subcore_axis_name="s")
    ones = jnp.ones((C,), jnp.float32)
    zeros = jnp.zeros((NP,), jnp.float32)
    k = pl.kernel(
        _sc_degree_body,
        out_type=jax.ShapeDtypeStruct((NC, NP), jnp.float32),
        mesh=mesh,
        scratch_types=[
            pltpu.VMEM((K, C), jnp.int32),
            pltpu.VMEM((C,), jnp.float32),
            pltpu.VMEM_SHARED((NP,), jnp.float32),
        ],
    )
    return k(dst3, ones, zeros)


def _sc_spmm(ys, src3, dst3):
    mesh = plsc.VectorSubcoreMesh(core_axis_name="c", subcore_axis_name="s")
    zeros = jnp.zeros((NP, D), jnp.float32)
    k = pl.kernel(
        _sc_spmm_body,
        out_type=jax.ShapeDtypeStruct((NP, NC * D), jnp.float32),
        mesh=mesh,
        scratch_types=[
            pltpu.VMEM((K // 2, C), jnp.int32),
            pltpu.VMEM((K // 2, C), jnp.int32),
            pltpu.VMEM((C, D), jnp.float32),
            pltpu.VMEM((C, D), jnp.float32),
            pltpu.VMEM_SHARED((NP, D), jnp.float32),
            pltpu.SemaphoreType.DMA,
            pltpu.SemaphoreType.DMA,
        ],
    )
    return k(ys, src3, dst3, zeros)


def _dinv(d0_ref, d1_ref):
    deg = d0_ref[...] + d1_ref[...] + 1.0
    return lax.rsqrt(deg)


def _tc_mm_body(x_ref, w_ref, o_ref):
    o_ref[...] = jnp.dot(x_ref[...], w_ref[...],
                         preferred_element_type=jnp.float32)


def _tc_scale_body(y_ref, d0_ref, d1_ref, o_ref):
    o_ref[...] = y_ref[...] * _dinv(d0_ref, d1_ref)


def _tc_mid_body(p_ref, ys_ref, d0_ref, d1_ref, b_ref, w_ref, o_ref):
    dinv = _dinv(d0_ref, d1_ref)
    psum = p_ref[:, :D] + p_ref[:, D:]
    h = jnp.maximum(dinv * (psum + ys_ref[...]) + b_ref[...], 0.0)
    o_ref[...] = jnp.dot(h, w_ref[...], preferred_element_type=jnp.float32) * dinv


def _tc_final_body(q_ref, ys_ref, d0_ref, d1_ref, b_ref, o_ref):
    dinv = _dinv(d0_ref, d1_ref)
    full = dinv * (q_ref[:, :D] + q_ref[:, D:] + ys_ref[...])
    o_ref[...] = full[:, :o_ref.shape[1]] + b_ref[...]


def _row_blocked(d):
    return pl.BlockSpec((_BLK, d), lambda i: (i, 0))


def _full(shape):
    return pl.BlockSpec(shape, lambda i: tuple(0 for _ in shape))


def kernel(x, edge_index, W1, b1, W2, b2):
    npad = EP - E
    pad_src = jnp.arange(npad, dtype=jnp.int32)
    pad_dst = N + (jnp.arange(npad, dtype=jnp.int32) >> 5)
    src3 = jnp.concatenate([edge_index[0], pad_src]).reshape(NW, K, C)
    dst3 = jnp.concatenate([edge_index[1], pad_dst]).reshape(NW, K, C)
    d_hid = W1.shape[1]
    d_out = W2.shape[1]

    y1 = pl.pallas_call(
        _tc_mm_body,
        grid=(_GRID,),
        in_specs=[_row_blocked(x.shape[1]), _full(W1.shape)],
        out_specs=_row_blocked(d_hid),
        out_shape=jax.ShapeDtypeStruct((N, d_hid), jnp.float32),
    )(x, W1)

    degp = _sc_degree(dst3)
    d0 = degp[0].reshape(NP, 1)
    d1 = degp[1].reshape(NP, 1)

    ys1 = pl.pallas_call(
        _tc_scale_body,
        grid=(_GRID,),
        in_specs=[_row_blocked(d_hid), _row_blocked(1), _row_blocked(1)],
        out_specs=_row_blocked(d_hid),
        out_shape=jax.ShapeDtypeStruct((N, d_hid), jnp.float32),
    )(y1, d0, d1)

    p = _sc_spmm(ys1, src3, dst3)

    W2p = jnp.zeros((d_hid, d_hid), jnp.float32).at[:, :d_out].set(W2)

    ys2 = pl.pallas_call(
        _tc_mid_body,
        grid=(_GRID,),
        in_specs=[_row_blocked(NC * D), _row_blocked(d_hid),
                  _row_blocked(1), _row_blocked(1),
                  _full((1, d_hid)), _full(W2p.shape)],
        out_specs=_row_blocked(d_hid),
        out_shape=jax.ShapeDtypeStruct((N, d_hid), jnp.float32),
    )(p, ys1, d0, d1, b1.reshape(1, d_hid), W2p)

    q = _sc_spmm(ys2, src3, dst3)

    out = pl.pallas_call(
        _tc_final_body,
        grid=(_GRID,),
        in_specs=[_row_blocked(NC * D), _row_blocked(d_hid),
                  _row_blocked(1), _row_blocked(1), _full((1, d_out))],
        out_specs=_row_blocked(d_out),
        out_shape=jax.ShapeDtypeStruct((N, d_out), jnp.float32),
    )(q, ys2, d0, d1, b2.reshape(1, d_out))

    return out


# speedup vs baseline: 31.2959x; 1.0175x over previous
import jax
import jax.numpy as jnp
from jax import lax
from jax.experimental import pallas as pl
from jax.experimental.pallas import tpu as pltpu
from jax.experimental.pallas import tpu_sc as plsc

N = 10000
E = 320000
NC = 2
NS = 16
NW = NC * NS
C = 128
K = 80
EP = NW * K * C
NP = 10240
ROWS_PER_TILE = NP // NS
D = 128

_BLK = 1000
_GRID = N // _BLK


def _sc_degree_body(dst_hbm, ones_hbm, zeros_hbm, out_hbm, dst_v, ones_v, acc):
    c = lax.axis_index("c")
    s = lax.axis_index("s")
    wid = s * NC + c
    row0 = s * ROWS_PER_TILE
    pltpu.sync_copy(zeros_hbm.at[pl.ds(row0, ROWS_PER_TILE)],
                    acc.at[pl.ds(row0, ROWS_PER_TILE)])
    pltpu.sync_copy(dst_hbm.at[wid], dst_v)
    pltpu.sync_copy(ones_hbm, ones_v)
    plsc.subcore_barrier()

    def step(j, carry):
        pltpu.sync_copy(ones_v, acc.at[dst_v.at[j]], add=True)
        return carry

    lax.fori_loop(0, K, step, 0)
    plsc.subcore_barrier()
    pltpu.sync_copy(acc.at[pl.ds(row0, ROWS_PER_TILE)],
                    out_hbm.at[c, pl.ds(row0, ROWS_PER_TILE)])


def _sc_spmm_body(ys_hbm, src_hbm, dst_hbm, zeros_hbm, out_hbm,
                  src_v, dst_v, rows_a, rows_b, acc, sem_a, sem_b):
    c = lax.axis_index("c")
    s = lax.axis_index("s")
    wid = s * NC + c
    row0 = s * ROWS_PER_TILE
    pltpu.sync_copy(zeros_hbm.at[pl.ds(row0, ROWS_PER_TILE)],
                    acc.at[pl.ds(row0, ROWS_PER_TILE)])
    plsc.subcore_barrier()

    def gather(j, buf, sem):
        return pltpu.make_async_copy(ys_hbm.at[src_v.at[j]], buf, sem)

    KH = K // 2
    for h in range(2):
        pltpu.sync_copy(src_hbm.at[wid, pl.ds(h * KH, KH)], src_v)
        pltpu.sync_copy(dst_hbm.at[wid, pl.ds(h * KH, KH)], dst_v)
        gather(0, rows_a, sem_a).start()

        def step(jj, carry):
            ja = 2 * jj
            gather(ja + 1, rows_b, sem_b).start()
            gather(ja, rows_a, sem_a).wait()
            pltpu.sync_copy(rows_a, acc.at[dst_v.at[ja]], add=True)

            @pl.when(jj < KH // 2 - 1)
            def _():
                gather(ja + 2, rows_a, sem_a).start()

            gather(ja + 1, rows_b, sem_b).wait()
            pltpu.sync_copy(rows_b, acc.at[dst_v.at[ja + 1]], add=True)
            return carry

        lax.fori_loop(0, KH // 2, step, 0)
    plsc.subcore_barrier()
    pltpu.sync_copy(acc.at[pl.ds(row0, ROWS_PER_TILE)],
                    out_hbm.at[pl.ds(row0, ROWS_PER_TILE), pl.ds(c * D, D)])


def _sc_degree(dst3):
    mesh = plsc.VectorSubcoreMesh(core_axis_name="c", subcore_axis_name="s")
    ones = jnp.ones((C,), jnp.float32)
    zeros = jnp.zeros((NP,), jnp.float32)
    k = pl.kernel(
        _sc_degree_body,
        out_type=jax.ShapeDtypeStruct((NC, NP), jnp.float32),
        mesh=mesh,
        scratch_types=[
            pltpu.VMEM((K, C), jnp.int32),
            pltpu.VMEM((C,), jnp.float32),
            pltpu.VMEM_SHARED((NP,), jnp.float32),
        ],
    )
    return k(dst3, ones, zeros)


def _sc_spmm(ys, src3, dst3):
    mesh = plsc.VectorSubcoreMesh(core_axis_name="c", subcore_axis_name="s")
    zeros = jnp.zeros((NP, D), jnp.float32)
    k = pl.kernel(
        _sc_spmm_body,
        out_type=jax.ShapeDtypeStruct((NP, NC * D), jnp.float32),
        mesh=mesh,
        scratch_types=[
            pltpu.VMEM((K // 2, C), jnp.int32),
            pltpu.VMEM((K // 2, C), jnp.int32),
            pltpu.VMEM((C, D), jnp.float32),
            pltpu.VMEM((C, D), jnp.float32),
            pltpu.VMEM_SHARED((NP, D), jnp.float32),
            pltpu.SemaphoreType.DMA,
            pltpu.SemaphoreType.DMA,
        ],
    )
    return k(ys, src3, dst3, zeros)


def _tc_mm_body(x_ref, w_ref, o_ref):
    o_ref[...] = jnp.dot(x_ref[...], w_ref[...],
                         preferred_element_type=jnp.float32)


def _tc_dinv_body(d_ref, o_ref):
    deg = d_ref[0:1, :] + d_ref[1:2, :] + 1.0
    dinv = lax.rsqrt(deg)
    o_ref[...] = jnp.broadcast_to(dinv, (128, o_ref.shape[0])).T


def _tc_scale_body(y_ref, dv_ref, o_ref):
    o_ref[...] = y_ref[...] * dv_ref[...]


def _tc_mid_body(p_ref, ys_ref, dv_ref, b_ref, w_ref, o_ref):
    dinv = dv_ref[...]
    psum = p_ref[:, :D] + p_ref[:, D:]
    h = jnp.maximum(dinv * (psum + ys_ref[...]) + b_ref[...], 0.0)
    o_ref[...] = jnp.dot(h, w_ref[...], preferred_element_type=jnp.float32) * dinv


def _tc_final_body(q_ref, ys_ref, dv_ref, b_ref, o_ref):
    full = dv_ref[...] * (q_ref[:, :D] + q_ref[:, D:] + ys_ref[...])
    o_ref[...] = full[:, :o_ref.shape[1]] + b_ref[...]


def _row_blocked(d):
    return pl.BlockSpec((_BLK, d), lambda i: (i, 0))


def _full(shape):
    return pl.BlockSpec(shape, lambda i: tuple(0 for _ in shape))


def kernel(x, edge_index, W1, b1, W2, b2):
    npad = EP - E
    pad_src = jnp.arange(npad, dtype=jnp.int32)
    pad_dst = N + (jnp.arange(npad, dtype=jnp.int32) >> 5)
    src3 = jnp.concatenate([edge_index[0], pad_src]).reshape(NW, K, C)
    dst3 = jnp.concatenate([edge_index[1], pad_dst]).reshape(NW, K, C)
    d_hid = W1.shape[1]
    d_out = W2.shape[1]

    y1 = pl.pallas_call(
        _tc_mm_body,
        grid=(_GRID,),
        in_specs=[_row_blocked(x.shape[1]), _full(W1.shape)],
        out_specs=_row_blocked(d_hid),
        out_shape=jax.ShapeDtypeStruct((N, d_hid), jnp.float32),
    )(x, W1)

    degp = _sc_degree(dst3)

    dinvb = pl.pallas_call(
        _tc_dinv_body,
        grid=(NP // 1280,),
        in_specs=[pl.BlockSpec((NC, 1280), lambda i: (0, i))],
        out_specs=pl.BlockSpec((1280, 128), lambda i: (i, 0)),
        out_shape=jax.ShapeDtypeStruct((NP, 128), jnp.float32),
    )(degp)

    ys1 = pl.pallas_call(
        _tc_scale_body,
        grid=(_GRID,),
        in_specs=[_row_blocked(d_hid), _row_blocked(128)],
        out_specs=_row_blocked(d_hid),
        out_shape=jax.ShapeDtypeStruct((N, d_hid), jnp.float32),
    )(y1, dinvb)

    p = _sc_spmm(ys1, src3, dst3)

    W2p = jnp.zeros((d_hid, d_hid), jnp.float32).at[:, :d_out].set(W2)

    ys2 = pl.pallas_call(
        _tc_mid_body,
        grid=(_GRID,),
        in_specs=[_row_blocked(NC * D), _row_blocked(d_hid),
                  _row_blocked(128),
                  _full((1, d_hid)), _full(W2p.shape)],
        out_specs=_row_blocked(d_hid),
        out_shape=jax.ShapeDtypeStruct((N, d_hid), jnp.float32),
    )(p, ys1, dinvb, b1.reshape(1, d_hid), W2p)

    q = _sc_spmm(ys2, src3, dst3)

    out = pl.pallas_call(
        _tc_final_body,
        grid=(_GRID,),
        in_specs=[_row_blocked(NC * D), _row_blocked(d_hid),
                  _row_blocked(128), _full((1, d_out))],
        out_specs=_row_blocked(d_out),
        out_shape=jax.ShapeDtypeStruct((N, d_out), jnp.float32),
    )(q, ys2, dinvb, b2.reshape(1, d_out))

    return out


# speedup vs baseline: 32.2198x; 1.0295x over previous
import jax
import jax.numpy as jnp
from jax import lax
from jax.experimental import pallas as pl
from jax.experimental.pallas import tpu as pltpu
from jax.experimental.pallas import tpu_sc as plsc

N = 10000
E = 320000
NC = 2
NS = 16
NW = NC * NS
C = 128
K = 80
EP = NW * K * C
NP = 10240
ROWS_PER_TILE = NP // NS
D = 128

_BLK = 2000
_GRID = N // _BLK


def _sc_degree_body(dst_hbm, ones_hbm, zeros_hbm, out_hbm, dst_v, ones_v, acc):
    c = lax.axis_index("c")
    s = lax.axis_index("s")
    wid = s * NC + c
    row0 = s * ROWS_PER_TILE
    pltpu.sync_copy(zeros_hbm.at[pl.ds(row0, ROWS_PER_TILE)],
                    acc.at[pl.ds(row0, ROWS_PER_TILE)])
    pltpu.sync_copy(dst_hbm.at[wid], dst_v)
    pltpu.sync_copy(ones_hbm, ones_v)
    plsc.subcore_barrier()

    def step(j, carry):
        pltpu.sync_copy(ones_v, acc.at[dst_v.at[j]], add=True)
        return carry

    lax.fori_loop(0, K, step, 0)
    plsc.subcore_barrier()
    pltpu.sync_copy(acc.at[pl.ds(row0, ROWS_PER_TILE)],
                    out_hbm.at[c, pl.ds(row0, ROWS_PER_TILE)])


def _sc_spmm_body(ys_hbm, src_hbm, dst_hbm, zeros_hbm, out_hbm,
                  src_v, dst_v, rows_a, rows_b, acc, sem_a, sem_b):
    c = lax.axis_index("c")
    s = lax.axis_index("s")
    wid = s * NC + c
    row0 = s * ROWS_PER_TILE

    def gather(j, buf, sem):
        return pltpu.make_async_copy(ys_hbm.at[src_v.at[j]], buf, sem)

    KH = K // 2
    pltpu.sync_copy(src_hbm.at[wid, pl.ds(0, KH)], src_v)
    pltpu.sync_copy(dst_hbm.at[wid, pl.ds(0, KH)], dst_v)
    gather(0, rows_a, sem_a).start()
    pltpu.sync_copy(zeros_hbm.at[pl.ds(row0, ROWS_PER_TILE)],
                    acc.at[pl.ds(row0, ROWS_PER_TILE)])
    plsc.subcore_barrier()
    for h in range(2):
        if h:
            pltpu.sync_copy(src_hbm.at[wid, pl.ds(h * KH, KH)], src_v)
            pltpu.sync_copy(dst_hbm.at[wid, pl.ds(h * KH, KH)], dst_v)
            gather(0, rows_a, sem_a).start()

        def step(jj, carry):
            ja = 2 * jj
            gather(ja + 1, rows_b, sem_b).start()
            gather(ja, rows_a, sem_a).wait()
            pltpu.sync_copy(rows_a, acc.at[dst_v.at[ja]], add=True)

            @pl.when(jj < KH // 2 - 1)
            def _():
                gather(ja + 2, rows_a, sem_a).start()

            gather(ja + 1, rows_b, sem_b).wait()
            pltpu.sync_copy(rows_b, acc.at[dst_v.at[ja + 1]], add=True)
            return carry

        lax.fori_loop(0, KH // 2, step, 0)
    plsc.subcore_barrier()
    pltpu.sync_copy(acc.at[pl.ds(row0, ROWS_PER_TILE)],
                    out_hbm.at[pl.ds(row0, ROWS_PER_TILE), pl.ds(c * D, D)])


def _sc_degree(dst3):
    mesh = plsc.VectorSubcoreMesh(core_axis_name="c", subcore_axis_name="s")
    ones = jnp.ones((C,), jnp.float32)
    zeros = jnp.zeros((NP,), jnp.float32)
    k = pl.kernel(
        _sc_degree_body,
        out_type=jax.ShapeDtypeStruct((NC, NP), jnp.float32),
        mesh=mesh,
        scratch_types=[
            pltpu.VMEM((K, C), jnp.int32),
            pltpu.VMEM((C,), jnp.float32),
            pltpu.VMEM_SHARED((NP,), jnp.float32),
        ],
    )
    return k(dst3, ones, zeros)


def _sc_spmm(ys, src3, dst3):
    mesh = plsc.VectorSubcoreMesh(core_axis_name="c", subcore_axis_name="s")
    zeros = jnp.zeros((NP, D), jnp.float32)
    k = pl.kernel(
        _sc_spmm_body,
        out_type=jax.ShapeDtypeStruct((NP, NC * D), jnp.float32),
        mesh=mesh,
        scratch_types=[
            pltpu.VMEM((K // 2, C), jnp.int32),
            pltpu.VMEM((K // 2, C), jnp.int32),
            pltpu.VMEM((C, D), jnp.float32),
            pltpu.VMEM((C, D), jnp.float32),
            pltpu.VMEM_SHARED((NP, D), jnp.float32),
            pltpu.SemaphoreType.DMA,
            pltpu.SemaphoreType.DMA,
        ],
    )
    return k(ys, src3, dst3, zeros)


def _tc_mm_body(x_ref, w_ref, o_ref):
    o_ref[...] = jnp.dot(x_ref[...], w_ref[...],
                         preferred_element_type=jnp.float32)


def _tc_dinv_body(d_ref, o_ref):
    deg = d_ref[0:1, :] + d_ref[1:2, :] + 1.0
    dinv = lax.rsqrt(deg)
    o_ref[...] = jnp.broadcast_to(dinv, (128, o_ref.shape[0])).T


def _tc_scale_body(y_ref, dv_ref, o_ref):
    o_ref[...] = y_ref[...] * dv_ref[...]


def _tc_mid_body(p_ref, ys_ref, dv_ref, b_ref, w_ref, o_ref):
    dinv = dv_ref[...]
    psum = p_ref[:, :D] + p_ref[:, D:]
    h = jnp.maximum(dinv * (psum + ys_ref[...]) + b_ref[...], 0.0)
    o_ref[...] = jnp.dot(h, w_ref[...], preferred_element_type=jnp.float32) * dinv


def _tc_final_body(q_ref, ys_ref, dv_ref, b_ref, o_ref):
    full = dv_ref[...] * (q_ref[:, :D] + q_ref[:, D:] + ys_ref[...])
    o_ref[...] = full[:, :o_ref.shape[1]] + b_ref[...]


def _row_blocked(d):
    return pl.BlockSpec((_BLK, d), lambda i: (i, 0))


def _full(shape):
    return pl.BlockSpec(shape, lambda i: tuple(0 for _ in shape))


def kernel(x, edge_index, W1, b1, W2, b2):
    npad = EP - E
    pad_src = jnp.arange(npad, dtype=jnp.int32)
    pad_dst = N + (jnp.arange(npad, dtype=jnp.int32) >> 5)
    src3 = jnp.concatenate([edge_index[0], pad_src]).reshape(NW, K, C)
    dst3 = jnp.concatenate([edge_index[1], pad_dst]).reshape(NW, K, C)
    d_hid = W1.shape[1]
    d_out = W2.shape[1]

    y1 = pl.pallas_call(
        _tc_mm_body,
        grid=(_GRID,),
        in_specs=[_row_blocked(x.shape[1]), _full(W1.shape)],
        out_specs=_row_blocked(d_hid),
        out_shape=jax.ShapeDtypeStruct((N, d_hid), jnp.float32),
    )(x, W1)

    degp = _sc_degree(dst3)

    dinvb = pl.pallas_call(
        _tc_dinv_body,
        grid=(NP // 1280,),
        in_specs=[pl.BlockSpec((NC, 1280), lambda i: (0, i))],
        out_specs=pl.BlockSpec((1280, 128), lambda i: (i, 0)),
        out_shape=jax.ShapeDtypeStruct((NP, 128), jnp.float32),
    )(degp)

    ys1 = pl.pallas_call(
        _tc_scale_body,
        grid=(_GRID,),
        in_specs=[_row_blocked(d_hid), _row_blocked(128)],
        out_specs=_row_blocked(d_hid),
        out_shape=jax.ShapeDtypeStruct((N, d_hid), jnp.float32),
    )(y1, dinvb)

    p = _sc_spmm(ys1, src3, dst3)

    W2p = jnp.zeros((d_hid, d_hid), jnp.float32).at[:, :d_out].set(W2)

    ys2 = pl.pallas_call(
        _tc_mid_body,
        grid=(_GRID,),
        in_specs=[_row_blocked(NC * D), _row_blocked(d_hid),
                  _row_blocked(128),
                  _full((1, d_hid)), _full(W2p.shape)],
        out_specs=_row_blocked(d_hid),
        out_shape=jax.ShapeDtypeStruct((N, d_hid), jnp.float32),
    )(p, ys1, dinvb, b1.reshape(1, d_hid), W2p)

    q = _sc_spmm(ys2, src3, dst3)

    out = pl.pallas_call(
        _tc_final_body,
        grid=(_GRID,),
        in_specs=[_row_blocked(NC * D), _row_blocked(d_hid),
                  _row_blocked(128), _full((1, d_out))],
        out_specs=_row_blocked(d_out),
        out_shape=jax.ShapeDtypeStruct((N, d_out), jnp.float32),
    )(q, ys2, dinvb, b2.reshape(1, d_out))

    return out


# speedup vs baseline: 33.6649x; 1.0449x over previous
import jax
import jax.numpy as jnp
from jax import lax
from jax.experimental import pallas as pl
from jax.experimental.pallas import tpu as pltpu
from jax.experimental.pallas import tpu_sc as plsc

N = 10000
E = 320000
NC = 2
NS = 16
NW = NC * NS
C = 128
NR = E // C
K = 80
W_ROWS = 44
W_BASE = NR - W_ROWS
W_OFF = (NW - 1) * K - W_BASE
NP = 10240
ROWS_PER_TILE = NP // NS
D = 128

_BLK = 2000
_GRID = N // _BLK


def _sc_degree_body(ei_hbm, ones_hbm, zeros_hbm, out_hbm, dst_v, ones_v, acc):
    c = lax.axis_index("c")
    s = lax.axis_index("s")
    wid = s * NC + c
    last = wid == NW - 1
    trips = jnp.where(last, NR - (NW - 1) * K, K)
    off = jnp.where(last, W_OFF, 0)
    row0 = s * ROWS_PER_TILE
    pltpu.sync_copy(zeros_hbm.at[pl.ds(row0, ROWS_PER_TILE)],
                    acc.at[pl.ds(row0, ROWS_PER_TILE)])

    @pl.when(jnp.logical_not(last))
    def _():
        pltpu.sync_copy(ei_hbm.at[1, pl.ds(wid * K, K)],
                        dst_v.at[pl.ds(0, K)])

    @pl.when(last)
    def _():
        pltpu.sync_copy(ei_hbm.at[1, pl.ds(W_BASE, W_ROWS)],
                        dst_v.at[pl.ds(0, W_ROWS)])

    pltpu.sync_copy(ones_hbm, ones_v)
    plsc.subcore_barrier()

    def step(j, carry):
        pltpu.sync_copy(ones_v, acc.at[dst_v.at[off + j]], add=True)
        return carry

    lax.fori_loop(0, trips, step, 0)
    plsc.subcore_barrier()
    pltpu.sync_copy(acc.at[pl.ds(row0, ROWS_PER_TILE)],
                    out_hbm.at[c, pl.ds(row0, ROWS_PER_TILE)])


def _sc_spmm_body(ys_hbm, ei_hbm, zeros_hbm, out_hbm,
                  src_v, dst_v, rows_a, rows_b, acc, sem_a, sem_b):
    c = lax.axis_index("c")
    s = lax.axis_index("s")
    wid = s * NC + c
    last = wid == NW - 1
    not_last = jnp.logical_not(last)
    row0 = s * ROWS_PER_TILE
    KH = K // 2
    LP = (NR - (NW - 1) * K) // 2

    def gather(j, buf, sem):
        return pltpu.make_async_copy(ys_hbm.at[src_v.at[j]], buf, sem)

    def loop(pairs, off):
        def step(jj, carry):
            ja = off + 2 * jj
            gather(ja + 1, rows_b, sem_b).start()
            gather(ja, rows_a, sem_a).wait()
            pltpu.sync_copy(rows_a, acc.at[dst_v.at[ja]], add=True)

            @pl.when(jj < pairs - 1)
            def _():
                gather(ja + 2, rows_a, sem_a).start()

            gather(ja + 1, rows_b, sem_b).wait()
            pltpu.sync_copy(rows_b, acc.at[dst_v.at[ja + 1]], add=True)
            return carry

        lax.fori_loop(0, pairs, step, 0)

    @pl.when(not_last)
    def _():
        pltpu.sync_copy(ei_hbm.at[0, pl.ds(wid * K, KH)],
                        src_v.at[pl.ds(0, KH)])
        pltpu.sync_copy(ei_hbm.at[1, pl.ds(wid * K, KH)],
                        dst_v.at[pl.ds(0, KH)])
        gather(0, rows_a, sem_a).start()

    @pl.when(last)
    def _():
        pltpu.sync_copy(ei_hbm.at[0, pl.ds(W_BASE, W_ROWS)],
                        src_v.at[pl.ds(0, W_ROWS)])
        pltpu.sync_copy(ei_hbm.at[1, pl.ds(W_BASE, W_ROWS)],
                        dst_v.at[pl.ds(0, W_ROWS)])
        gather(W_OFF, rows_a, sem_a).start()

    @pl.when(c == 0)
    def _():
        pltpu.sync_copy(ys_hbm.at[pl.ds(row0, ROWS_PER_TILE)],
                        acc.at[pl.ds(row0, ROWS_PER_TILE)])

    @pl.when(c == 1)
    def _():
        pltpu.sync_copy(zeros_hbm.at[pl.ds(row0, ROWS_PER_TILE)],
                        acc.at[pl.ds(row0, ROWS_PER_TILE)])

    plsc.subcore_barrier()

    @pl.when(not_last)
    def _():
        loop(KH // 2, 0)
        pltpu.sync_copy(ei_hbm.at[0, pl.ds(wid * K + KH, KH)],
                        src_v.at[pl.ds(0, KH)])
        pltpu.sync_copy(ei_hbm.at[1, pl.ds(wid * K + KH, KH)],
                        dst_v.at[pl.ds(0, KH)])
        gather(0, rows_a, sem_a).start()
        loop(KH // 2, 0)

    @pl.when(last)
    def _():
        loop(LP, W_OFF)

    plsc.subcore_barrier()
    pltpu.sync_copy(acc.at[pl.ds(row0, ROWS_PER_TILE)],
                    out_hbm.at[pl.ds(row0, ROWS_PER_TILE), pl.ds(c * D, D)])


def _sc_degree(ei3):
    mesh = plsc.VectorSubcoreMesh(core_axis_name="c", subcore_axis_name="s")
    ones = jnp.ones((C,), jnp.float32)
    zeros = jnp.zeros((NP,), jnp.float32)
    k = pl.kernel(
        _sc_degree_body,
        out_type=jax.ShapeDtypeStruct((NC, NP), jnp.float32),
        mesh=mesh,
        scratch_types=[
            pltpu.VMEM((K, C), jnp.int32),
            pltpu.VMEM((C,), jnp.float32),
            pltpu.VMEM_SHARED((NP,), jnp.float32),
        ],
    )
    return k(ei3, ones, zeros)


def _sc_spmm(ys, ei3):
    mesh = plsc.VectorSubcoreMesh(core_axis_name="c", subcore_axis_name="s")
    zeros = jnp.zeros((NP, D), jnp.float32)
    k = pl.kernel(
        _sc_spmm_body,
        out_type=jax.ShapeDtypeStruct((NP, NC * D), jnp.float32),
        mesh=mesh,
        scratch_types=[
            pltpu.VMEM((W_ROWS, C), jnp.int32),
            pltpu.VMEM((W_ROWS, C), jnp.int32),
            pltpu.VMEM((C, D), jnp.float32),
            pltpu.VMEM((C, D), jnp.float32),
            pltpu.VMEM_SHARED((NP, D), jnp.float32),
            pltpu.SemaphoreType.DMA,
            pltpu.SemaphoreType.DMA,
        ],
    )
    return k(ys, ei3, zeros)


def _tc_mm_body(x_ref, w_ref, o_ref):
    o_ref[...] = jnp.dot(x_ref[...], w_ref[...],
                         preferred_element_type=jnp.float32)


def _tc_dinv_body(d_ref, o_ref):
    deg = d_ref[0:1, :] + d_ref[1:2, :] + 1.0
    dinv = lax.rsqrt(deg)
    o_ref[...] = jnp.broadcast_to(dinv, (128, o_ref.shape[0])).T


def _tc_scale_body(y_ref, dv_ref, o_ref):
    o_ref[...] = y_ref[...] * dv_ref[...]


def _tc_mid_body(p_ref, dv_ref, b_ref, w_ref, o_ref):
    dinv = dv_ref[...]
    psum = p_ref[:, :D] + p_ref[:, D:]
    h = jnp.maximum(dinv * psum + b_ref[...], 0.0)
    o_ref[...] = jnp.dot(h, w_ref[...], preferred_element_type=jnp.float32) * dinv


def _tc_final_body(q_ref, dv_ref, b_ref, o_ref):
    full = dv_ref[...] * (q_ref[:, :D] + q_ref[:, D:])
    o_ref[...] = full[:, :o_ref.shape[1]] + b_ref[...]


def _row_blocked(d):
    return pl.BlockSpec((_BLK, d), lambda i: (i, 0))


def _full(shape):
    return pl.BlockSpec(shape, lambda i: tuple(0 for _ in shape))


def kernel(x, edge_index, W1, b1, W2, b2):
    ei3 = edge_index.reshape(2, NR, C)
    d_hid = W1.shape[1]
    d_out = W2.shape[1]

    y1 = pl.pallas_call(
        _tc_mm_body,
        grid=(_GRID,),
        in_specs=[_row_blocked(x.shape[1]), _full(W1.shape)],
        out_specs=_row_blocked(d_hid),
        out_shape=jax.ShapeDtypeStruct((N, d_hid), jnp.float32),
    )(x, W1)

    degp = _sc_degree(ei3)

    dinvb = pl.pallas_call(
        _tc_dinv_body,
        grid=(NP // 1280,),
        in_specs=[pl.BlockSpec((NC, 1280), lambda i: (0, i))],
        out_specs=pl.BlockSpec((1280, 128), lambda i: (i, 0)),
        out_shape=jax.ShapeDtypeStruct((NP, 128), jnp.float32),
    )(degp)

    ys1 = pl.pallas_call(
        _tc_scale_body,
        grid=(_GRID,),
        in_specs=[_row_blocked(d_hid), _row_blocked(128)],
        out_specs=_row_blocked(d_hid),
        out_shape=jax.ShapeDtypeStruct((NP, d_hid), jnp.float32),
    )(y1, dinvb)

    p = _sc_spmm(ys1, ei3)

    W2p = jnp.zeros((d_hid, d_hid), jnp.float32).at[:, :d_out].set(W2)

    ys2 = pl.pallas_call(
        _tc_mid_body,
        grid=(_GRID,),
        in_specs=[_row_blocked(NC * D), _row_blocked(128),
                  _full((1, d_hid)), _full(W2p.shape)],
        out_specs=_row_blocked(d_hid),
        out_shape=jax.ShapeDtypeStruct((NP, d_hid), jnp.float32),
    )(p, dinvb, b1.reshape(1, d_hid), W2p)

    q = _sc_spmm(ys2, ei3)

    out = pl.pallas_call(
        _tc_final_body,
        grid=(_GRID,),
        in_specs=[_row_blocked(NC * D),
                  _row_blocked(128), _full((1, d_out))],
        out_specs=_row_blocked(d_out),
        out_shape=jax.ShapeDtypeStruct((N, d_out), jnp.float32),
    )(q, dinvb, b2.reshape(1, d_out))

    return out
